# Initial kernel scaffold; baseline (speedup 1.0000x reference)
import jax
import jax.numpy as jnp
from jax.experimental import pallas as pl


def kernel(x, prototypes, feat_g, feat_b, proto_g, proto_b, mask_g, mask_b):
    raise NotImplementedError("write your pallas kernel here")


# speedup vs baseline: 1.2422x; 1.2422x over previous
import jax
import jax.numpy as jnp
from jax.experimental import pallas as pl
from jax.experimental.pallas import tpu as pltpu

_NUM_CLASSES = 19
_NUM_PROTO = 10
_D = 768
_P = _NUM_CLASSES * _NUM_PROTO
_BN = 1024


def _proto_prep_body(w_ref, wn_ref):
    w = w_ref[:]
    norm = jnp.sqrt(jnp.sum(w * w, axis=0, keepdims=True))
    wn_ref[:] = w / (norm + 1e-12)


def _main_body(x_ref, w_ref, fg_ref, fb_ref, pg_ref, pb_ref, mg_ref, mb_ref,
               o_ref):
    x = x_ref[:]
    mu = jnp.mean(x, axis=1, keepdims=True)
    xc = x - mu
    var = jnp.mean(xc * xc, axis=1, keepdims=True)
    c = xc / jnp.sqrt(var + 1e-5) * fg_ref[:] + fb_ref[:]
    n2 = jnp.sqrt(jnp.sum(c * c, axis=1, keepdims=True))
    c = c / (n2 + 1e-12)
    sim = jnp.dot(c, w_ref[:], preferred_element_type=jnp.float32)
    mu2 = jnp.mean(sim, axis=1, keepdims=True)
    s2 = sim - mu2
    var2 = jnp.mean(s2 * s2, axis=1, keepdims=True)
    s = s2 / jnp.sqrt(var2 + 1e-5) * pg_ref[:] + pb_ref[:]
    out = s[:, 0:_NUM_CLASSES]
    for m in range(1, _NUM_PROTO):
        out = jnp.maximum(out, s[:, m * _NUM_CLASSES:(m + 1) * _NUM_CLASSES])
    mu3 = jnp.mean(out, axis=1, keepdims=True)
    o2 = out - mu3
    var3 = jnp.mean(o2 * o2, axis=1, keepdims=True)
    o_ref[:] = o2 / jnp.sqrt(var3 + 1e-5) * mg_ref[:] + mb_ref[:]


@jax.jit
def _run(x, prototypes, feat_g, feat_b, proto_g, proto_b, mask_g, mask_b):
    wt = prototypes.transpose(1, 0, 2).reshape(_P, _D).T
    wn = pl.pallas_call(
        _proto_prep_body,
        out_shape=jax.ShapeDtypeStruct((_D, _P), jnp.float32),
    )(wt)
    pg = proto_g.reshape(_NUM_CLASSES, _NUM_PROTO).T.reshape(1, _P)
    pb = proto_b.reshape(_NUM_CLASSES, _NUM_PROTO).T.reshape(1, _P)
    n = x.shape[0]
    grid = n // _BN
    const = lambda i: (0, 0)
    out = pl.pallas_call(
        _main_body,
        grid=(grid,),
        in_specs=[
            pl.BlockSpec((_BN, _D), lambda i: (i, 0)),
            pl.BlockSpec((_D, _P), const),
            pl.BlockSpec((1, _D), const),
            pl.BlockSpec((1, _D), const),
            pl.BlockSpec((1, _P), const),
            pl.BlockSpec((1, _P), const),
            pl.BlockSpec((1, _NUM_CLASSES), const),
            pl.BlockSpec((1, _NUM_CLASSES), const),
        ],
        out_specs=pl.BlockSpec((_BN, _NUM_CLASSES), lambda i: (i, 0)),
        out_shape=jax.ShapeDtypeStruct((n, _NUM_CLASSES), jnp.float32),
        compiler_params=pltpu.CompilerParams(
            dimension_semantics=("parallel",)),
    )(x, wn, feat_g.reshape(1, _D), feat_b.reshape(1, _D), pg, pb,
      mask_g.reshape(1, _NUM_CLASSES), mask_b.reshape(1, _NUM_CLASSES))
    return out


def kernel(x, prototypes, feat_g, feat_b, proto_g, proto_b, mask_g, mask_b):
    return _run(x, prototypes, feat_g, feat_b, proto_g, proto_b,
                mask_g, mask_b)
